# baseline (speedup 1.0000x reference)
import functools

import jax
import jax.numpy as jnp
from jax import lax
from jax.experimental import pallas as pl
from jax.experimental.pallas import tpu as pltpu
from jax.experimental.pallas import tpu_sc as plsc

N_A = 10000
M_N = 16
AF = 64
NBRF = 41
ORIG = 92
NE = N_A * M_N
NCRY = 8
NC = 128
TWO_AF = 2 * AF
EPS = 1e-5
TN = 1000
GRID_N = N_A // TN
_PREC = lax.Precision.HIGHEST
_F32 = jnp.float32


def _softplus(x):
    return jnp.maximum(x, 0.) + jnp.log(1. + jnp.exp(-jnp.abs(x)))


def _make_sc_gather(D, Btot):
    info = plsc.get_sparse_core_info()
    ncr, nsc = info.num_cores, info.num_subcores
    nw = ncr * nsc
    assert Btot % nw == 0
    b_per_w = Btot // nw
    ch = min(128, b_per_w)
    n_full = b_per_w // ch
    tail = b_per_w - n_full * ch
    nbuf = min(4, n_full)
    ngrp = (n_full + nbuf - 1) // nbuf
    mesh = plsc.VectorSubcoreMesh(core_axis_name="c", subcore_axis_name="s")
    scratch = [pltpu.VMEM((b_per_w,), jnp.int32)]
    scratch += [pltpu.VMEM((ch, D), _F32) for _ in range(nbuf)]
    scratch += [pltpu.SemaphoreType.DMA for _ in range(2 * nbuf)]
    if tail:
        scratch += [pltpu.VMEM((tail, D), _F32), pltpu.SemaphoreType.DMA]

    def body(table_hbm, idx_hbm, out_hbm, idx_all, *rest):
        rows = rest[:nbuf]
        sg = rest[nbuf:2 * nbuf]
        sw = rest[2 * nbuf:3 * nbuf]
        ts = rest[3 * nbuf:]
        wid = lax.axis_index("s") * ncr + lax.axis_index("c")
        base = wid * b_per_w
        pltpu.sync_copy(idx_hbm.at[pl.ds(base, b_per_w)], idx_all)

        def g_desc(i, b):
            return pltpu.make_async_copy(
                table_hbm.at[idx_all.at[pl.ds(i * ch, ch)]], rows[b], sg[b])

        def w_desc(i, b):
            return pltpu.make_async_copy(
                rows[b], out_hbm.at[pl.ds(base + i * ch, ch)], sw[b])

        for b in range(nbuf):
            if b < n_full:
                g_desc(b, b).start()

        def group(g, c):
            for b in range(nbuf):
                i = g * nbuf + b

                @pl.when(i < n_full)
                def _():
                    g_desc(i, b).wait()
                    w_desc(i, b).start()
                    nxt = i + nbuf

                    @pl.when(nxt < n_full)
                    def _():
                        w_desc(i, b).wait()
                        g_desc(nxt, b).start()
            return c

        lax.fori_loop(0, ngrp, group, 0)
        for b in range(nbuf):
            if b < n_full:
                i_last = n_full - 1 - ((n_full - 1 - b) % nbuf)
                w_desc(i_last, b).wait()
        if tail:
            rows_t, sem_t = ts
            off = n_full * ch
            pltpu.async_copy(
                table_hbm.at[idx_all.at[pl.ds(off, tail)]], rows_t,
                sem_t).wait()
            pltpu.sync_copy(rows_t, out_hbm.at[pl.ds(base + off, tail)])

    return pl.kernel(
        body,
        out_type=jax.ShapeDtypeStruct((Btot, D), _F32),
        mesh=mesh,
        scratch_types=scratch,
    )


_gather_edges = _make_sc_gather(TWO_AF, NE)
_gather_cry = _make_sc_gather(TWO_AF, NCRY * NC)


def _dot(a, b, prec=_PREC):
    return jnp.dot(a, b, precision=prec, preferred_element_type=_F32)


def _embed_body(a_ref, ew_ref, ws_ref, b_ref, wn_ref, xo_ref, po_ref, to_ref):
    x = _dot(a_ref[...], ew_ref[...])
    xo_ref[...] = x
    po_ref[...] = _dot(x, ws_ref[...]) + b_ref[...]
    to_ref[...] = _dot(x, wn_ref[...])


def _edge_stats_body(gx_ref, nb_ref, p_ref, we_ref, out_ref):
    nb = nb_ref[...].reshape(TN * M_N, NBRF)
    g = gx_ref[...] + _dot(nb, we_ref[...], lax.Precision.DEFAULT)
    g3 = g.reshape(TN, M_N, TWO_AF) + p_ref[...][:, None, :]
    s1 = jnp.sum(g3, axis=(0, 1))[None]
    s2 = jnp.sum(g3 * g3, axis=(0, 1))[None]
    acc = jnp.concatenate([s1, s2], axis=0)

    @pl.when(pl.program_id(0) == 0)
    def _():
        out_ref[...] = jnp.zeros_like(out_ref)

    out_ref[...] += acc


def _edge_apply_body(gx_ref, nb_ref, p_ref, we_ref, st_ref, g1_ref,
                     b1_ref, ns_ref, s2_ref):
    mu = st_ref[0:1, :] * (1.0 / NE)
    var = st_ref[1:2, :] * (1.0 / NE) - mu * mu
    sc = g1_ref[...] * lax.rsqrt(var + EPS)
    sh = b1_ref[...] - mu * sc
    nb = nb_ref[...].reshape(TN * M_N, NBRF)
    e_s = _dot(nb, we_ref[...] * sc, lax.Precision.DEFAULT)
    base = p_ref[...] * sc + sh
    y = gx_ref[...] * sc + e_s
    z = y.reshape(TN, M_N, TWO_AF) + base[:, None, :]
    filt = 0.5 + 0.5 * jnp.tanh(0.5 * z[..., :AF])
    core = _softplus(z[..., AF:])
    ns = jnp.sum(filt * core, axis=1)
    ns_ref[...] = ns
    p1 = jnp.sum(ns, axis=0)[None]
    p2 = jnp.sum(ns * ns, axis=0)[None]
    acc = jnp.concatenate([p1, p2], axis=0)

    @pl.when(pl.program_id(0) == 0)
    def _():
        s2_ref[...] = jnp.zeros_like(s2_ref)

    s2_ref[...] += acc


def _bn2_x(x_ref, ns_ref, st_ref, g2_ref, b2_ref):
    mu = st_ref[0:1, :] * (1.0 / N_A)
    var = st_ref[1:2, :] * (1.0 / N_A) - mu * mu
    sc = g2_ref[...] * lax.rsqrt(var + EPS)
    sh = b2_ref[...] - mu * sc
    return _softplus(x_ref[...] + ns_ref[...] * sc + sh)


def _epi_proj_body(x_ref, ns_ref, st_ref, g2_ref, b2_ref, ws_ref, b_ref,
                   wn_ref, xo_ref, po_ref, to_ref):
    xn = _bn2_x(x_ref, ns_ref, st_ref, g2_ref, b2_ref)
    xo_ref[...] = xn
    po_ref[...] = _dot(xn, ws_ref[...]) + b_ref[...]
    to_ref[...] = _dot(xn, wn_ref[...])


def _epi_final_body(x_ref, ns_ref, st_ref, g2_ref, b2_ref, xo_ref):
    xn = _bn2_x(x_ref, ns_ref, st_ref, g2_ref, b2_ref)
    xo_ref[...] = jnp.concatenate(
        [xn, jnp.zeros((N_A, TWO_AF - AF), _F32)], axis=1)


def _dec_body(af_ref, bw_ref, bb_ref, f1w_ref, f1b_ref, fw_ref, fb_ref,
              ep_ref, ao_ref):
    af = af_ref[0][:, :AF]
    mb = _dot(af, bw_ref[...])
    cols = [jnp.sum(mb[:, o * AF:(o + 1) * AF] * af, axis=1, keepdims=True)
            for o in range(6)]
    q = jnp.concatenate(cols, axis=1) + bb_ref[...]
    y = _dot(q, f1w_ref[...]) + f1b_ref[...]
    m = jnp.max(y, axis=1, keepdims=True)
    lse = jnp.log(jnp.sum(jnp.exp(y - m), axis=1, keepdims=True))
    lg = y - m - lse
    ep_ref[0] = jnp.broadcast_to(lg[None], (NC, NC, 6))
    ao_ref[0] = _dot(af, fw_ref[...]) + fb_ref[...]


_embed_call = pl.pallas_call(
    _embed_body,
    out_shape=(jax.ShapeDtypeStruct((N_A, AF), _F32),
               jax.ShapeDtypeStruct((N_A, TWO_AF), _F32),
               jax.ShapeDtypeStruct((N_A, TWO_AF), _F32)),
)

_full = lambda shape: pl.BlockSpec(shape, lambda i: tuple(0 for _ in shape))

_edge_in_specs = [
    pl.BlockSpec((TN * M_N, TWO_AF), lambda i: (i, 0)),
    pl.BlockSpec((TN, M_N, NBRF), lambda i: (i, 0, 0)),
    pl.BlockSpec((TN, TWO_AF), lambda i: (i, 0)),
    _full((NBRF, TWO_AF)),
]

_edge_stats_call = pl.pallas_call(
    _edge_stats_body,
    grid=(GRID_N,),
    in_specs=_edge_in_specs,
    out_specs=_full((2, TWO_AF)),
    out_shape=jax.ShapeDtypeStruct((2, TWO_AF), _F32),
)

_edge_apply_call = pl.pallas_call(
    _edge_apply_body,
    grid=(GRID_N,),
    in_specs=_edge_in_specs + [
        _full((2, TWO_AF)),
        _full((1, TWO_AF)),
        _full((1, TWO_AF)),
    ],
    out_specs=[
        pl.BlockSpec((TN, AF), lambda i: (i, 0)),
        _full((2, AF)),
    ],
    out_shape=(jax.ShapeDtypeStruct((N_A, AF), _F32),
               jax.ShapeDtypeStruct((2, AF), _F32)),
)

_epi_proj_call = pl.pallas_call(
    _epi_proj_body,
    out_shape=(jax.ShapeDtypeStruct((N_A, AF), _F32),
               jax.ShapeDtypeStruct((N_A, TWO_AF), _F32),
               jax.ShapeDtypeStruct((N_A, TWO_AF), _F32)),
)

_epi_final_call = pl.pallas_call(
    _epi_final_body,
    out_shape=jax.ShapeDtypeStruct((N_A, TWO_AF), _F32),
)

_dec_call = pl.pallas_call(
    _dec_body,
    grid=(NCRY,),
    in_specs=[
        pl.BlockSpec((1, NC, TWO_AF), lambda i: (i, 0, 0)),
        _full((AF, 6 * AF)),
        _full((1, 6)),
        _full((6, 6)),
        _full((1, 6)),
        _full((AF, ORIG)),
        _full((1, ORIG)),
    ],
    out_specs=[
        pl.BlockSpec((1, NC, NC, 6), lambda i: (i, 0, 0, 0)),
        pl.BlockSpec((1, NC, ORIG), lambda i: (i, 0, 0)),
    ],
    out_shape=(jax.ShapeDtypeStruct((NCRY, NC, NC, 6), _F32),
               jax.ShapeDtypeStruct((NCRY, NC, ORIG), _F32)),
)


def kernel(atom_fea, nbr_fea, nbr_fea_idx, crystal_atom_idx, emb_w,
           fc_w_0, fc_b_0, bn1_g_0, bn1_b_0, bn2_g_0, bn2_b_0,
           fc_w_1, fc_b_1, bn1_g_1, bn1_b_1, bn2_g_1, bn2_b_1,
           fc_w_2, fc_b_2, bn1_g_2, bn1_b_2, bn2_g_2, bn2_b_2,
           bil_w, bil_b, fc1_w, fc1_b, fcaf_w, fcaf_b):
    idx = nbr_fea_idx.astype(jnp.int32).reshape(NE)
    cidx = crystal_atom_idx.astype(jnp.int32).reshape(NCRY * NC)

    convs = [
        (fc_w_0, fc_b_0, bn1_g_0, bn1_b_0, bn2_g_0, bn2_b_0),
        (fc_w_1, fc_b_1, bn1_g_1, bn1_b_1, bn2_g_1, bn2_b_1),
        (fc_w_2, fc_b_2, bn1_g_2, bn1_b_2, bn2_g_2, bn2_b_2),
    ]
    ws_t = [w[:, :AF].T for (w, *_r) in convs]
    wn_t = [w[:, AF:2 * AF].T for (w, *_r) in convs]
    we_t = [w[:, 2 * AF:].T for (w, *_r) in convs]
    fb = [b[None] for (_w, b, *_r) in convs]

    x, p, tab = _embed_call(atom_fea, emb_w.T, ws_t[0], fb[0], wn_t[0])
    for i in range(3):
        _w, _b, g1, b1, g2, b2 = convs[i]
        gx = _gather_edges(tab, idx)
        st = _edge_stats_call(gx, nbr_fea, p, we_t[i])
        ns, st2 = _edge_apply_call(gx, nbr_fea, p, we_t[i], st,
                                   g1[None], b1[None])
        if i < 2:
            x, p, tab = _epi_proj_call(x, ns, st2, g2[None], b2[None],
                                       ws_t[i + 1], fb[i + 1], wn_t[i + 1])
        else:
            xpad = _epi_final_call(x, ns, st2, g2[None], b2[None])

    af = _gather_cry(xpad, cidx)
    bilcat = jnp.transpose(bil_w, (1, 0, 2)).reshape(AF, 6 * AF)
    ep4, ao = _dec_call(af.reshape(NCRY, NC, TWO_AF), bilcat, bil_b[None],
                        fc1_w.T, fc1_b[None], fcaf_w.T, fcaf_b[None])
    return ep4.reshape(NCRY, NC * NC, 6), ao


# speedup vs baseline: 1.0756x; 1.0756x over previous
import functools

import jax
import jax.numpy as jnp
from jax import lax
from jax.experimental import pallas as pl
from jax.experimental.pallas import tpu as pltpu
from jax.experimental.pallas import tpu_sc as plsc

N_A = 10000
M_N = 16
AF = 64
NBRF = 41
ORIG = 92
NE = N_A * M_N
NCRY = 8
NC = 128
TWO_AF = 2 * AF
EPS = 1e-5
TN = 800
GRID_N = N_A // TN
_PREC = lax.Precision.HIGHEST
_F32 = jnp.float32


def _softplus(x):
    return jnp.maximum(x, 0.) + jnp.log(1. + jnp.exp(-jnp.abs(x)))


def _make_sc_gather(D, Btot):
    info = plsc.get_sparse_core_info()
    ncr, nsc = info.num_cores, info.num_subcores
    nw = ncr * nsc
    assert Btot % nw == 0
    b_per_w = Btot // nw
    ch = min(128, b_per_w)
    n_full = b_per_w // ch
    tail = b_per_w - n_full * ch
    nbuf = min(4, n_full)
    ngrp = (n_full + nbuf - 1) // nbuf
    mesh = plsc.VectorSubcoreMesh(core_axis_name="c", subcore_axis_name="s")
    scratch = [pltpu.VMEM((b_per_w,), jnp.int32)]
    scratch += [pltpu.VMEM((ch, D), _F32) for _ in range(nbuf)]
    scratch += [pltpu.SemaphoreType.DMA for _ in range(2 * nbuf)]
    if tail:
        scratch += [pltpu.VMEM((tail, D), _F32), pltpu.SemaphoreType.DMA]

    def body(table_hbm, idx_hbm, out_hbm, idx_all, *rest):
        rows = rest[:nbuf]
        sg = rest[nbuf:2 * nbuf]
        sw = rest[2 * nbuf:3 * nbuf]
        ts = rest[3 * nbuf:]
        wid = lax.axis_index("s") * ncr + lax.axis_index("c")
        base = wid * b_per_w
        pltpu.sync_copy(idx_hbm.at[pl.ds(base, b_per_w)], idx_all)

        def g_desc(i, b):
            return pltpu.make_async_copy(
                table_hbm.at[idx_all.at[pl.ds(i * ch, ch)]], rows[b], sg[b])

        def w_desc(i, b):
            return pltpu.make_async_copy(
                rows[b], out_hbm.at[pl.ds(base + i * ch, ch)], sw[b])

        for b in range(nbuf):
            if b < n_full:
                g_desc(b, b).start()

        def group(g, c):
            for b in range(nbuf):
                i = g * nbuf + b

                @pl.when(i < n_full)
                def _():
                    g_desc(i, b).wait()
                    w_desc(i, b).start()
                    nxt = i + nbuf

                    @pl.when(nxt < n_full)
                    def _():
                        w_desc(i, b).wait()
                        g_desc(nxt, b).start()
            return c

        lax.fori_loop(0, ngrp, group, 0)
        for b in range(nbuf):
            if b < n_full:
                i_last = n_full - 1 - ((n_full - 1 - b) % nbuf)
                w_desc(i_last, b).wait()
        if tail:
            rows_t, sem_t = ts
            off = n_full * ch
            pltpu.async_copy(
                table_hbm.at[idx_all.at[pl.ds(off, tail)]], rows_t,
                sem_t).wait()
            pltpu.sync_copy(rows_t, out_hbm.at[pl.ds(base + off, tail)])

    return pl.kernel(
        body,
        out_type=jax.ShapeDtypeStruct((Btot, D), _F32),
        mesh=mesh,
        scratch_types=scratch,
    )


_gather_edges = _make_sc_gather(TWO_AF, NE)
_gather_cry = _make_sc_gather(TWO_AF, NCRY * NC)


def _dot(a, b, prec=_PREC):
    return jnp.dot(a, b, precision=prec, preferred_element_type=_F32)


def _embed_body(a_ref, ew_ref, ws_ref, b_ref, wn_ref, xo_ref, po_ref, to_ref):
    x = _dot(a_ref[...], ew_ref[...])
    xo_ref[...] = x
    po_ref[...] = _dot(x, ws_ref[...], lax.Precision.DEFAULT) + b_ref[...]
    to_ref[...] = _dot(x, wn_ref[...], lax.Precision.DEFAULT)


def _edge_stats_body(gx_ref, nb_ref, p_ref, we_ref, out_ref):
    nb = nb_ref[...].reshape(TN * M_N, NBRF)
    g = gx_ref[...] + _dot(nb, we_ref[...], lax.Precision.DEFAULT)
    g3 = g.reshape(TN, M_N, TWO_AF) + p_ref[...][:, None, :]
    s1 = jnp.sum(g3, axis=(0, 1))[None]
    s2 = jnp.sum(g3 * g3, axis=(0, 1))[None]
    acc = jnp.concatenate([s1, s2], axis=0)

    @pl.when(pl.program_id(0) == 0)
    def _():
        out_ref[...] = jnp.zeros_like(out_ref)

    out_ref[...] += acc


def _edge_apply_body(gx_ref, nb_ref, p_ref, we_ref, st_ref, g1_ref,
                     b1_ref, ns_ref, s2_ref):
    mu = st_ref[0:1, :] * (1.0 / NE)
    var = st_ref[1:2, :] * (1.0 / NE) - mu * mu
    sc = g1_ref[...] * lax.rsqrt(var + EPS)
    sh = b1_ref[...] - mu * sc
    nb = nb_ref[...].reshape(TN * M_N, NBRF)
    e_s = _dot(nb, we_ref[...] * sc, lax.Precision.DEFAULT)
    base = p_ref[...] * sc + sh
    y = gx_ref[...] * sc + e_s
    z = y.reshape(TN, M_N, TWO_AF) + base[:, None, :]
    filt = 0.5 + 0.5 * jnp.tanh(0.5 * z[..., :AF])
    core = _softplus(z[..., AF:])
    ns = jnp.sum(filt * core, axis=1)
    ns_ref[...] = ns
    p1 = jnp.sum(ns, axis=0)[None]
    p2 = jnp.sum(ns * ns, axis=0)[None]
    acc = jnp.concatenate([p1, p2], axis=0)

    @pl.when(pl.program_id(0) == 0)
    def _():
        s2_ref[...] = jnp.zeros_like(s2_ref)

    s2_ref[...] += acc


def _bn2_x(x_ref, ns_ref, st_ref, g2_ref, b2_ref):
    mu = st_ref[0:1, :] * (1.0 / N_A)
    var = st_ref[1:2, :] * (1.0 / N_A) - mu * mu
    sc = g2_ref[...] * lax.rsqrt(var + EPS)
    sh = b2_ref[...] - mu * sc
    return _softplus(x_ref[...] + ns_ref[...] * sc + sh)


def _epi_proj_body(x_ref, ns_ref, st_ref, g2_ref, b2_ref, ws_ref, b_ref,
                   wn_ref, xo_ref, po_ref, to_ref):
    xn = _bn2_x(x_ref, ns_ref, st_ref, g2_ref, b2_ref)
    xo_ref[...] = xn
    po_ref[...] = _dot(xn, ws_ref[...], lax.Precision.DEFAULT) + b_ref[...]
    to_ref[...] = _dot(xn, wn_ref[...], lax.Precision.DEFAULT)


def _epi_final_body(x_ref, ns_ref, st_ref, g2_ref, b2_ref, xo_ref):
    xn = _bn2_x(x_ref, ns_ref, st_ref, g2_ref, b2_ref)
    xo_ref[...] = jnp.concatenate(
        [xn, jnp.zeros((N_A, TWO_AF - AF), _F32)], axis=1)


def _dec_body(af_ref, bw_ref, bb_ref, f1w_ref, f1b_ref, fw_ref, fb_ref,
              ep_ref, ao_ref):
    af = af_ref[0][:, :AF]
    mb = _dot(af, bw_ref[...])
    cols = [jnp.sum(mb[:, o * AF:(o + 1) * AF] * af, axis=1, keepdims=True)
            for o in range(6)]
    q = jnp.concatenate(cols, axis=1) + bb_ref[...]
    y = _dot(q, f1w_ref[...]) + f1b_ref[...]
    m = jnp.max(y, axis=1, keepdims=True)
    lse = jnp.log(jnp.sum(jnp.exp(y - m), axis=1, keepdims=True))
    lg = y - m - lse
    ep_ref[0] = jnp.broadcast_to(lg[None], (NC, NC, 6))
    ao_ref[0] = _dot(af, fw_ref[...]) + fb_ref[...]


_embed_call = pl.pallas_call(
    _embed_body,
    out_shape=(jax.ShapeDtypeStruct((N_A, AF), _F32),
               jax.ShapeDtypeStruct((N_A, TWO_AF), _F32),
               jax.ShapeDtypeStruct((N_A, TWO_AF), _F32)),
)

_full = lambda shape: pl.BlockSpec(shape, lambda i: tuple(0 for _ in shape))

_edge_in_specs = [
    pl.BlockSpec((TN * M_N, TWO_AF), lambda i: (i, 0)),
    pl.BlockSpec((TN, M_N, NBRF), lambda i: (i, 0, 0)),
    pl.BlockSpec((TN, TWO_AF), lambda i: (i, 0)),
    _full((NBRF, TWO_AF)),
]

_edge_stats_call = pl.pallas_call(
    _edge_stats_body,
    grid=(GRID_N,),
    in_specs=_edge_in_specs,
    out_specs=_full((2, TWO_AF)),
    out_shape=jax.ShapeDtypeStruct((2, TWO_AF), _F32),
)

_edge_apply_call = pl.pallas_call(
    _edge_apply_body,
    grid=(GRID_N,),
    in_specs=_edge_in_specs + [
        _full((2, TWO_AF)),
        _full((1, TWO_AF)),
        _full((1, TWO_AF)),
    ],
    out_specs=[
        pl.BlockSpec((TN, AF), lambda i: (i, 0)),
        _full((2, AF)),
    ],
    out_shape=(jax.ShapeDtypeStruct((N_A, AF), _F32),
               jax.ShapeDtypeStruct((2, AF), _F32)),
)

_epi_proj_call = pl.pallas_call(
    _epi_proj_body,
    out_shape=(jax.ShapeDtypeStruct((N_A, AF), _F32),
               jax.ShapeDtypeStruct((N_A, TWO_AF), _F32),
               jax.ShapeDtypeStruct((N_A, TWO_AF), _F32)),
)

_epi_final_call = pl.pallas_call(
    _epi_final_body,
    out_shape=jax.ShapeDtypeStruct((N_A, TWO_AF), _F32),
)

_dec_call = pl.pallas_call(
    _dec_body,
    grid=(NCRY,),
    in_specs=[
        pl.BlockSpec((1, NC, TWO_AF), lambda i: (i, 0, 0)),
        _full((AF, 6 * AF)),
        _full((1, 6)),
        _full((6, 6)),
        _full((1, 6)),
        _full((AF, ORIG)),
        _full((1, ORIG)),
    ],
    out_specs=[
        pl.BlockSpec((1, NC, NC, 6), lambda i: (i, 0, 0, 0)),
        pl.BlockSpec((1, NC, ORIG), lambda i: (i, 0, 0)),
    ],
    out_shape=(jax.ShapeDtypeStruct((NCRY, NC, NC, 6), _F32),
               jax.ShapeDtypeStruct((NCRY, NC, ORIG), _F32)),
)


def kernel(atom_fea, nbr_fea, nbr_fea_idx, crystal_atom_idx, emb_w,
           fc_w_0, fc_b_0, bn1_g_0, bn1_b_0, bn2_g_0, bn2_b_0,
           fc_w_1, fc_b_1, bn1_g_1, bn1_b_1, bn2_g_1, bn2_b_1,
           fc_w_2, fc_b_2, bn1_g_2, bn1_b_2, bn2_g_2, bn2_b_2,
           bil_w, bil_b, fc1_w, fc1_b, fcaf_w, fcaf_b):
    idx = nbr_fea_idx.astype(jnp.int32).reshape(NE)
    cidx = crystal_atom_idx.astype(jnp.int32).reshape(NCRY * NC)

    convs = [
        (fc_w_0, fc_b_0, bn1_g_0, bn1_b_0, bn2_g_0, bn2_b_0),
        (fc_w_1, fc_b_1, bn1_g_1, bn1_b_1, bn2_g_1, bn2_b_1),
        (fc_w_2, fc_b_2, bn1_g_2, bn1_b_2, bn2_g_2, bn2_b_2),
    ]
    ws_t = [w[:, :AF].T for (w, *_r) in convs]
    wn_t = [w[:, AF:2 * AF].T for (w, *_r) in convs]
    we_t = [w[:, 2 * AF:].T for (w, *_r) in convs]
    fb = [b[None] for (_w, b, *_r) in convs]

    x, p, tab = _embed_call(atom_fea, emb_w.T, ws_t[0], fb[0], wn_t[0])
    for i in range(3):
        _w, _b, g1, b1, g2, b2 = convs[i]
        gx = _gather_edges(tab, idx)
        st = _edge_stats_call(gx, nbr_fea, p, we_t[i])
        ns, st2 = _edge_apply_call(gx, nbr_fea, p, we_t[i], st,
                                   g1[None], b1[None])
        if i < 2:
            x, p, tab = _epi_proj_call(x, ns, st2, g2[None], b2[None],
                                       ws_t[i + 1], fb[i + 1], wn_t[i + 1])
        else:
            xpad = _epi_final_call(x, ns, st2, g2[None], b2[None])

    af = _gather_cry(xpad, cidx)
    bilcat = jnp.transpose(bil_w, (1, 0, 2)).reshape(AF, 6 * AF)
    ep4, ao = _dec_call(af.reshape(NCRY, NC, TWO_AF), bilcat, bil_b[None],
                        fc1_w.T, fc1_b[None], fcaf_w.T, fcaf_b[None])
    return ep4.reshape(NCRY, NC * NC, 6), ao
